# baseline (device time: 61867 ns/iter reference)
import jax
import jax.numpy as jnp
from jax import lax
from jax.experimental import pallas as pl
from jax.experimental.pallas import tpu as pltpu

N_DEV = 8
B, SQ, SKV = 2, 256, 256
HQ_LOCAL, DH = 4, 64
D_MODEL = 512


def kernel(x, Wq, K_ext, V_ext, Wo):
    my = lax.axis_index("i")
    h0 = my * HQ_LOCAL
    K_loc = jnp.transpose(
        lax.dynamic_slice_in_dim(K_ext, h0, HQ_LOCAL, axis=2), (0, 2, 1, 3)
    )
    V_loc = jnp.transpose(
        lax.dynamic_slice_in_dim(V_ext, h0, HQ_LOCAL, axis=2), (0, 2, 1, 3)
    )

    def body(x_ref, wq_ref, k_ref, v_ref, wo_ref, out_ref,
             comm_ref, send_sems, recv_sems):
        my_pos = lax.axis_index("i")
        left = lax.rem(my_pos + N_DEV - 1, N_DEV)
        right = lax.rem(my_pos + 1, N_DEV)

        barrier_sem = pltpu.get_barrier_semaphore()
        for nbr in (left, right):
            pl.semaphore_signal(
                barrier_sem, inc=1,
                device_id=(nbr,), device_id_type=pl.DeviceIdType.MESH,
            )
        pl.semaphore_wait(barrier_sem, 2)

        wq = wq_ref[...].astype(jnp.bfloat16)
        wo = wo_ref[...].astype(jnp.bfloat16)
        qi = lax.broadcasted_iota(jnp.int32, (SQ, SKV), 0)
        ki = lax.broadcasted_iota(jnp.int32, (SQ, SKV), 1)
        mask = (jnp.abs(qi - ki) <= 128) | (ki < 32) | (qi < 32)

        for b in range(B):
            xb = x_ref[b].astype(jnp.bfloat16)
            q = jnp.dot(xb, wq, preferred_element_type=jnp.float32)
            ctx_parts = []
            for h in range(HQ_LOCAL):
                qh = q[:, h * DH:(h + 1) * DH].astype(jnp.bfloat16)
                kh = k_ref[b, h].astype(jnp.bfloat16)
                s = lax.dot_general(
                    qh, kh, (((1,), (1,)), ((), ())),
                    preferred_element_type=jnp.float32,
                ) * 0.125
                s = jnp.where(mask, s, -1e9)
                m = jnp.max(s, axis=-1, keepdims=True)
                w = jnp.exp(s - m)
                w = w / jnp.sum(w, axis=-1, keepdims=True)
                vh = v_ref[b, h].astype(jnp.bfloat16)
                ctx_parts.append(jnp.dot(
                    w.astype(jnp.bfloat16), vh,
                    preferred_element_type=jnp.float32,
                ))
            ctx = jnp.concatenate(ctx_parts, axis=-1).astype(jnp.bfloat16)
            part = jnp.dot(ctx, wo, preferred_element_type=jnp.float32)
            out_ref[b] = part
            comm_ref[0, b] = part.astype(jnp.bfloat16)

        for h in range(N_DEV - 1):
            rdma = pltpu.make_async_remote_copy(
                src_ref=comm_ref.at[h],
                dst_ref=comm_ref.at[h + 1],
                send_sem=send_sems.at[h],
                recv_sem=recv_sems.at[h],
                device_id=(right,),
                device_id_type=pl.DeviceIdType.MESH,
            )
            rdma.start()
            rdma.wait()
            out_ref[...] = out_ref[...] + comm_ref[h + 1].astype(jnp.float32)

    return pl.pallas_call(
        body,
        out_shape=jax.ShapeDtypeStruct((B, SQ, D_MODEL), jnp.float32),
        in_specs=[pl.BlockSpec(memory_space=pltpu.VMEM)] * 5,
        out_specs=pl.BlockSpec(memory_space=pltpu.VMEM),
        scratch_shapes=[
            pltpu.VMEM((N_DEV, B, SQ, D_MODEL), jnp.bfloat16),
            pltpu.SemaphoreType.DMA((N_DEV - 1,)),
            pltpu.SemaphoreType.DMA((N_DEV - 1,)),
        ],
        compiler_params=pltpu.CompilerParams(collective_id=0),
    )(x, Wq, K_loc, V_loc, Wo)


# device time: 20742 ns/iter; 2.9827x vs baseline; 2.9827x over previous
import jax
import jax.numpy as jnp
from jax import lax
from jax.experimental import pallas as pl
from jax.experimental.pallas import tpu as pltpu

N_DEV = 8
B, SQ, SKV = 2, 256, 256
HQ_LOCAL, DH = 4, 64
D_MODEL = 512
RQ = SQ // N_DEV


def kernel(x, Wq, K_ext, V_ext, Wo):
    my = lax.axis_index("i")
    h0 = my * HQ_LOCAL
    K_loc = jnp.transpose(
        lax.dynamic_slice_in_dim(K_ext, h0, HQ_LOCAL, axis=2), (0, 2, 1, 3)
    )
    V_loc = jnp.transpose(
        lax.dynamic_slice_in_dim(V_ext, h0, HQ_LOCAL, axis=2), (0, 2, 1, 3)
    )

    def body(x_ref, wq_ref, k_ref, v_ref, wo_ref, out_ref,
             send_buf, recv_buf, red_buf, s1, r1, s2, r2):
        my_pos = lax.axis_index("i")

        barrier_sem = pltpu.get_barrier_semaphore()
        for o in range(1, N_DEV):
            pl.semaphore_signal(
                barrier_sem, inc=1,
                device_id=(lax.rem(my_pos + o, N_DEV),),
                device_id_type=pl.DeviceIdType.MESH,
            )
        pl.semaphore_wait(barrier_sem, N_DEV - 1)

        wq = wq_ref[...].astype(jnp.bfloat16)
        wo = wo_ref[...].astype(jnp.bfloat16)
        qi = lax.broadcasted_iota(jnp.int32, (SQ, SKV), 0)
        ki = lax.broadcasted_iota(jnp.int32, (SQ, SKV), 1)
        mask = (jnp.abs(qi - ki) <= 128) | (ki < 32) | (qi < 32)

        for b in range(B):
            xb = x_ref[b].astype(jnp.bfloat16)
            q = jnp.dot(xb, wq, preferred_element_type=jnp.float32)
            ctx_parts = []
            for h in range(HQ_LOCAL):
                qh = q[:, h * DH:(h + 1) * DH].astype(jnp.bfloat16)
                kh = k_ref[b, h].astype(jnp.bfloat16)
                s = lax.dot_general(
                    qh, kh, (((1,), (1,)), ((), ())),
                    preferred_element_type=jnp.float32,
                ) * 0.125
                s = jnp.where(mask, s, -1e9)
                m = jnp.max(s, axis=-1, keepdims=True)
                w = jnp.exp(s - m)
                w = w / jnp.sum(w, axis=-1, keepdims=True)
                vh = v_ref[b, h].astype(jnp.bfloat16)
                ctx_parts.append(jnp.dot(
                    w.astype(jnp.bfloat16), vh,
                    preferred_element_type=jnp.float32,
                ))
            ctx = jnp.concatenate(ctx_parts, axis=-1).astype(jnp.bfloat16)
            part = jnp.dot(ctx, wo, preferred_element_type=jnp.float32)
            for t in range(N_DEV):
                send_buf[t, b] = part[t * RQ:(t + 1) * RQ, :].astype(
                    jnp.bfloat16)

        p1 = []
        for o in range(1, N_DEV):
            tgt = lax.rem(my_pos + o, N_DEV)
            rdma = pltpu.make_async_remote_copy(
                src_ref=send_buf.at[tgt],
                dst_ref=recv_buf.at[my_pos],
                send_sem=s1.at[o],
                recv_sem=r1.at[my_pos],
                device_id=(tgt,),
                device_id_type=pl.DeviceIdType.MESH,
            )
            rdma.start()
            p1.append(rdma)

        acc = send_buf[my_pos].astype(jnp.float32)
        for o in range(1, N_DEV):
            src = lax.rem(my_pos + N_DEV - o, N_DEV)
            recv = pltpu.make_async_remote_copy(
                src_ref=send_buf.at[0],
                dst_ref=recv_buf.at[src],
                send_sem=s1.at[o],
                recv_sem=r1.at[src],
                device_id=(src,),
                device_id_type=pl.DeviceIdType.MESH,
            )
            recv.wait_recv()
            acc = acc + recv_buf[src].astype(jnp.float32)

        red_buf[...] = acc.astype(jnp.bfloat16)
        out_ref[:, pl.ds(my_pos * RQ, RQ), :] = red_buf[...]
        p2 = []
        for o in range(1, N_DEV):
            tgt = lax.rem(my_pos + o, N_DEV)
            rdma = pltpu.make_async_remote_copy(
                src_ref=red_buf,
                dst_ref=out_ref.at[:, pl.ds(my_pos * RQ, RQ), :],
                send_sem=s2.at[o],
                recv_sem=r2.at[my_pos],
                device_id=(tgt,),
                device_id_type=pl.DeviceIdType.MESH,
            )
            rdma.start()
            p2.append(rdma)

        for o in range(1, N_DEV):
            src = lax.rem(my_pos + N_DEV - o, N_DEV)
            recv = pltpu.make_async_remote_copy(
                src_ref=red_buf,
                dst_ref=out_ref.at[:, pl.ds(src * RQ, RQ), :],
                send_sem=s2.at[o],
                recv_sem=r2.at[src],
                device_id=(src,),
                device_id_type=pl.DeviceIdType.MESH,
            )
            recv.wait_recv()

        for rdma in p1 + p2:
            rdma.wait_send()

    return pl.pallas_call(
        body,
        out_shape=jax.ShapeDtypeStruct((B, SQ, D_MODEL), jnp.bfloat16),
        in_specs=[pl.BlockSpec(memory_space=pltpu.VMEM)] * 5,
        out_specs=pl.BlockSpec(memory_space=pltpu.VMEM),
        scratch_shapes=[
            pltpu.VMEM((N_DEV, B, RQ, D_MODEL), jnp.bfloat16),
            pltpu.VMEM((N_DEV, B, RQ, D_MODEL), jnp.bfloat16),
            pltpu.VMEM((B, RQ, D_MODEL), jnp.bfloat16),
            pltpu.SemaphoreType.DMA((N_DEV,)),
            pltpu.SemaphoreType.DMA((N_DEV,)),
            pltpu.SemaphoreType.DMA((N_DEV,)),
            pltpu.SemaphoreType.DMA((N_DEV,)),
        ],
        compiler_params=pltpu.CompilerParams(collective_id=0),
    )(x, Wq, K_loc, V_loc, Wo)
